# baseline (speedup 1.0000x reference)
import functools

import jax
import jax.numpy as jnp
from jax import lax
from jax.experimental import pallas as pl
from jax.experimental.pallas import tpu as pltpu
from jax.experimental.pallas import tpu_sc as plsc

_CHUNK = 128


def _build_gather(n_workers, n_chunks, d_model):
    mesh = plsc.VectorSubcoreMesh(core_axis_name="c", subcore_axis_name="s")
    n_rows = n_chunks * _CHUNK
    num_cores = 2

    @functools.partial(
        pl.kernel,
        mesh=mesh,
        compiler_params=pltpu.CompilerParams(use_tc_tiling_on_sc=False),
        out_type=jax.ShapeDtypeStruct((n_workers * n_rows, d_model), jnp.float32),
        scratch_types=[
            pltpu.VMEM((n_chunks, _CHUNK), jnp.int32),
            pltpu.VMEM((_CHUNK, d_model), jnp.float32),
            pltpu.SemaphoreType.DMA,
        ],
    )
    def gather_kernel(table_hbm, idx_hbm, out_hbm, idx_v, rows_v, sem):
        wid = lax.axis_index("s") * num_cores + lax.axis_index("c")
        base = wid * n_rows
        pltpu.sync_copy(idx_hbm.at[wid], idx_v)

        def body(j, carry):
            pltpu.async_copy(table_hbm.at[idx_v.at[j]], rows_v, sem).wait()
            pltpu.sync_copy(rows_v, out_hbm.at[pl.ds(base + j * _CHUNK, _CHUNK)])
            return carry

        lax.fori_loop(0, n_chunks, body, 0)

    return gather_kernel


def kernel(token_ids, weight):
    b, s = token_ids.shape
    d_model = weight.shape[1]
    total = b * s
    flat = token_ids.reshape(-1).astype(jnp.int32)

    n_workers = 32
    per_worker = -(-total // (n_workers * _CHUNK)) * _CHUNK
    padded = n_workers * per_worker
    if padded != total:
        flat = jnp.concatenate(
            [flat, jnp.zeros((padded - total,), jnp.int32)], axis=0
        )
    idx3 = flat.reshape(n_workers, per_worker // _CHUNK, _CHUNK)

    gather = _build_gather(n_workers, per_worker // _CHUNK, d_model)
    out = gather(weight, idx3)
    return out[:total].reshape(b, s, d_model)


# speedup vs baseline: 1.1163x; 1.1163x over previous
import functools

import jax
import jax.numpy as jnp
from jax import lax
from jax.experimental import pallas as pl
from jax.experimental.pallas import tpu as pltpu
from jax.experimental.pallas import tpu_sc as plsc

_CHUNK = 128
_NBUF = 4


def _build_gather(n_workers, n_chunks, d_model):
    mesh = plsc.VectorSubcoreMesh(core_axis_name="c", subcore_axis_name="s")
    n_rows = n_chunks * _CHUNK
    n_groups = n_chunks // _NBUF
    num_cores = 2

    @functools.partial(
        pl.kernel,
        mesh=mesh,
        compiler_params=pltpu.CompilerParams(use_tc_tiling_on_sc=False),
        out_type=jax.ShapeDtypeStruct((n_workers * n_rows, d_model), jnp.float32),
        scratch_types=[
            pltpu.VMEM((n_chunks, _CHUNK), jnp.int32),
            pltpu.VMEM((_NBUF, _CHUNK, d_model), jnp.float32),
            pltpu.SemaphoreType.DMA((_NBUF,)),
        ],
    )
    def gather_kernel(table_hbm, idx_hbm, out_hbm, idx_v, rows_v, gsem):
        wid = lax.axis_index("s") * num_cores + lax.axis_index("c")
        base = wid * n_rows
        pltpu.sync_copy(idx_hbm.at[wid], idx_v)

        def start(j, b):
            pltpu.async_copy(table_hbm.at[idx_v.at[j]], rows_v.at[b], gsem.at[b])

        def wait_and_store(j, b):
            pltpu.make_async_copy(
                table_hbm.at[pl.ds(0, _CHUNK)], rows_v.at[b], gsem.at[b]
            ).wait()
            pltpu.sync_copy(
                rows_v.at[b], out_hbm.at[pl.ds(base + j * _CHUNK, _CHUNK)]
            )

        for b in range(_NBUF):
            start(b, b)

        def body(g, carry):
            j0 = g * _NBUF
            for b in range(_NBUF):
                wait_and_store(j0 + b, b)
                start(j0 + b + _NBUF, b)
            return carry

        lax.fori_loop(0, n_groups - 1, body, 0)

        j0 = (n_groups - 1) * _NBUF
        for b in range(_NBUF):
            wait_and_store(j0 + b, b)

    return gather_kernel


def kernel(token_ids, weight):
    b, s = token_ids.shape
    d_model = weight.shape[1]
    total = b * s
    flat = token_ids.reshape(-1).astype(jnp.int32)

    n_workers = 32
    grain = n_workers * _CHUNK * _NBUF
    padded = -(-total // grain) * grain
    if padded != total:
        flat = jnp.concatenate(
            [flat, jnp.zeros((padded - total,), jnp.int32)], axis=0
        )
    per_worker = padded // n_workers
    idx3 = flat.reshape(n_workers, per_worker // _CHUNK, _CHUNK)

    gather = _build_gather(n_workers, per_worker // _CHUNK, d_model)
    out = gather(weight, idx3)
    return out[:total].reshape(b, s, d_model)
